# Initial kernel scaffold; baseline (speedup 1.0000x reference)
import jax
import jax.numpy as jnp
from jax.experimental import pallas as pl


def kernel(input_ids, labels, negative_samples, emb_in, emb_out):
    raise NotImplementedError("write your pallas kernel here")


# speedup vs baseline: 3.3830x; 3.3830x over previous
import dataclasses
import functools

import jax
import jax.numpy as jnp
from jax import lax
from jax.experimental import pallas as pl
from jax.experimental.pallas import tpu as pltpu
from jax.experimental.pallas import tpu_sc as plsc

LANES = 16
STREAM_IDX = 128


def _make_sc_call(B, CTX, NEG, D, dtype):
  mesh = plsc.VectorSubcoreMesh(core_axis_name="c", subcore_axis_name="s")
  NC = mesh.num_cores
  NW = NC * mesh.num_subcores
  assert B % (NW * LANES) == 0
  PER_W = B // NW
  C = 32
  NCHUNK = PER_W // C
  GROUPS = C // LANES

  def body(ctx_idx_hbm, tgt_idx_hbm, neg_idx_hbm, emb_in_hbm, emb_out_hbm,
           pos_hbm, neg_hbm,
           ctx_idx_v, tgt_idx_v, neg_idx_v,
           ctx_rows, tgt_rows, neg_rows,
           pos_buf, neg_buf, sem):
    cid = lax.axis_index("c")
    sid = lax.axis_index("s")
    wid = sid * NC + cid
    base = wid * PER_W

    pltpu.sync_copy(ctx_idx_hbm.at[pl.ds(base * CTX, PER_W * CTX)], ctx_idx_v)
    pltpu.sync_copy(tgt_idx_hbm.at[pl.ds(base, PER_W)], tgt_idx_v)
    pltpu.sync_copy(neg_idx_hbm.at[pl.ds(base * NEG, PER_W * NEG)], neg_idx_v)

    e_iota = lax.iota(jnp.int32, LANES)

    @pl.loop(0, NCHUNK)
    def _chunk(ci):
      co = ci * C
      copies = []
      for k in range(C * CTX // STREAM_IDX):
        copies.append(pltpu.async_copy(
            emb_in_hbm.at[ctx_idx_v.at[pl.ds(co * CTX + k * STREAM_IDX,
                                             STREAM_IDX)]],
            ctx_rows.at[pl.ds(k * STREAM_IDX, STREAM_IDX)], sem))
      for k in range(C * NEG // STREAM_IDX):
        copies.append(pltpu.async_copy(
            emb_out_hbm.at[neg_idx_v.at[pl.ds(co * NEG + k * STREAM_IDX,
                                              STREAM_IDX)]],
            neg_rows.at[pl.ds(k * STREAM_IDX, STREAM_IDX)], sem))
      copies.append(pltpu.async_copy(
          emb_out_hbm.at[tgt_idx_v.at[pl.ds(co, C)]], tgt_rows, sem))
      for cp in copies:
        cp.wait()

      for g in range(GROUPS):
        erow = e_iota + g * LANES
        ctx_base = erow * CTX
        neg_base = erow * NEG
        ctx_rows_j = [ctx_base + j for j in range(CTX)]
        neg_rows_n = [neg_base + n for n in range(NEG)]

        def dbody(d, carry, _ctx_rows_j=ctx_rows_j, _neg_rows_n=neg_rows_n,
                  _erow=erow):
          pos_acc, neg_accs = carry
          cold = jnp.full((LANES,), 0, jnp.int32) + d
          m = plsc.load_gather(ctx_rows, [_ctx_rows_j[0], cold])
          for j in range(1, CTX):
            m = m + plsc.load_gather(ctx_rows, [_ctx_rows_j[j], cold])
          m = m * (1.0 / CTX)
          t = plsc.load_gather(tgt_rows, [_erow, cold])
          pos_acc = pos_acc + t * m
          neg_accs = tuple(
              neg_accs[n] + plsc.load_gather(neg_rows, [_neg_rows_n[n], cold]) * m
              for n in range(NEG))
          return pos_acc, neg_accs

        zero = jnp.zeros((LANES,), jnp.float32)
        pos_acc, neg_accs = lax.fori_loop(0, D, dbody, (zero, (zero,) * NEG))

        pos_buf[pl.ds(co + g * LANES, LANES)] = pos_acc
        for n in range(NEG):
          plsc.store_scatter(neg_buf,
                             [co + erow, jnp.full((LANES,), n, jnp.int32)],
                             neg_accs[n])

    pltpu.sync_copy(pos_buf, pos_hbm.at[pl.ds(base, PER_W)])
    pltpu.sync_copy(neg_buf, neg_hbm.at[pl.ds(base, PER_W)])

  cp = pltpu.CompilerParams()
  fields = getattr(pltpu.CompilerParams, "__dataclass_fields__", {})
  if "needs_layout_passes" in fields:
    cp = dataclasses.replace(cp, needs_layout_passes=False)
  if "use_tc_tiling_on_sc" in fields:
    cp = dataclasses.replace(cp, use_tc_tiling_on_sc=False)

  return pl.kernel(
      body,
      out_type=(jax.ShapeDtypeStruct((B,), dtype),
                jax.ShapeDtypeStruct((B, NEG), dtype)),
      mesh=mesh,
      compiler_params=cp,
      scratch_types=[
          pltpu.VMEM((PER_W * CTX,), jnp.int32),
          pltpu.VMEM((PER_W,), jnp.int32),
          pltpu.VMEM((PER_W * NEG,), jnp.int32),
          pltpu.VMEM((C * CTX, D), dtype),
          pltpu.VMEM((C, D), dtype),
          pltpu.VMEM((C * NEG, D), dtype),
          pltpu.VMEM((PER_W,), dtype),
          pltpu.VMEM((PER_W, NEG), dtype),
          pltpu.SemaphoreType.DMA,
      ],
  )


def kernel(input_ids, labels, negative_samples, emb_in, emb_out):
  B, CTX = input_ids.shape
  NEG = negative_samples.shape[1]
  D = emb_in.shape[1]
  ctx_idx = input_ids.reshape(-1).astype(jnp.int32)
  tgt_idx = labels.reshape(-1).astype(jnp.int32)
  neg_idx = negative_samples.reshape(-1).astype(jnp.int32)
  call = _make_sc_call(B, CTX, NEG, D, emb_in.dtype)
  return call(ctx_idx, tgt_idx, neg_idx, emb_in, emb_out)


# speedup vs baseline: 3.4342x; 1.0151x over previous
import dataclasses
import functools

import jax
import jax.numpy as jnp
from jax import lax
from jax.experimental import pallas as pl
from jax.experimental.pallas import tpu as pltpu
from jax.experimental.pallas import tpu_sc as plsc

LANES = 16
STREAM_IDX = 128


def _tree_sum(vals):
  vals = list(vals)
  while len(vals) > 1:
    nxt = [a + b for a, b in zip(vals[0::2], vals[1::2])]
    if len(vals) % 2:
      nxt.append(vals[-1])
    vals = nxt
  return vals[0]


def _make_sc_call(B, CTX, NEG, D, dtype):
  mesh = plsc.VectorSubcoreMesh(core_axis_name="c", subcore_axis_name="s")
  NC = mesh.num_cores
  NW = NC * mesh.num_subcores
  assert B % (NW * LANES) == 0
  PER_W = B // NW
  C = LANES
  NCHUNK = PER_W // C
  assert NCHUNK % 2 == 0

  def body(ctx_idx_hbm, tgt_idx_hbm, neg_idx_hbm, emb_in_hbm, emb_out_hbm,
           pos_hbm, neg_hbm,
           ctx_idx_v, tgt_idx_v, neg_idx_v,
           ctx_rows, tgt_rows, neg_rows,
           mean_v, pos_buf, neg_buf, sems):
    cid = lax.axis_index("c")
    sid = lax.axis_index("s")
    wid = sid * NC + cid
    base = wid * PER_W

    pltpu.sync_copy(ctx_idx_hbm.at[pl.ds(base * CTX, PER_W * CTX)], ctx_idx_v)
    pltpu.sync_copy(tgt_idx_hbm.at[pl.ds(base, PER_W)], tgt_idx_v)
    pltpu.sync_copy(neg_idx_hbm.at[pl.ds(base * NEG, PER_W * NEG)], neg_idx_v)

    e_iota = lax.iota(jnp.int32, LANES)

    def stream_pieces(n):
      pieces, o = [], 0
      while o < n:
        w = min(STREAM_IDX, n - o)
        pieces.append((o, w))
        o += w
      return pieces

    def gather_descs(ci, b):
      descs = []
      co = ci * C
      for o, w in stream_pieces(C * CTX):
        descs.append(pltpu.make_async_copy(
            emb_in_hbm.at[ctx_idx_v.at[pl.ds(co * CTX + o, w)]],
            ctx_rows.at[b].at[pl.ds(o, w)], sems.at[b]))
      for o, w in stream_pieces(C * NEG):
        descs.append(pltpu.make_async_copy(
            emb_out_hbm.at[neg_idx_v.at[pl.ds(co * NEG + o, w)]],
            neg_rows.at[b].at[pl.ds(o, w)], sems.at[b]))
      descs.append(pltpu.make_async_copy(
          emb_out_hbm.at[tgt_idx_v.at[pl.ds(co, C)]],
          tgt_rows.at[b], sems.at[b]))
      return descs

    def issue(ci, b):
      for d_ in gather_descs(ci, b):
        d_.start()

    def drain(ci, b):
      for d_ in gather_descs(ci, b):
        d_.wait()

    row_ctx = [e_iota * CTX + j for j in range(CTX)]
    row_neg = [e_iota * NEG + n for n in range(NEG)]

    def compute(ci, b):
      co = ci * C
      crows = ctx_rows.at[b]
      nrows = neg_rows.at[b]
      trows = tgt_rows.at[b]

      @pl.loop(0, D)
      def _mean(d):
        cold = jnp.full((LANES,), 0, jnp.int32) + d
        m = _tree_sum([plsc.load_gather(crows, [row_ctx[j], cold])
                       for j in range(CTX)])
        mean_v[pl.ds(d * LANES, LANES)] = m * (1.0 / CTX)

      def dbody(d, carry):
        pos_acc, neg_accs = carry
        cold = jnp.full((LANES,), 0, jnp.int32) + d
        m = mean_v[pl.ds(d * LANES, LANES)]
        pos_acc = pos_acc + plsc.load_gather(trows, [e_iota, cold]) * m
        neg_accs = tuple(
            neg_accs[n] + plsc.load_gather(nrows, [row_neg[n], cold]) * m
            for n in range(NEG))
        return pos_acc, neg_accs

      zero = jnp.zeros((LANES,), jnp.float32)
      pos_acc, neg_accs = lax.fori_loop(0, D, dbody, (zero, (zero,) * NEG))

      pos_buf[pl.ds(co, LANES)] = pos_acc
      for n in range(NEG):
        plsc.store_scatter(neg_buf,
                           [co + e_iota, jnp.full((LANES,), n, jnp.int32)],
                           neg_accs[n])

    issue(0, 0)

    @pl.loop(0, NCHUNK, step=2)
    def _chunk(ci):
      issue(ci + 1, 1)
      drain(ci, 0)
      compute(ci, 0)

      @pl.when(ci + 2 < NCHUNK)
      def _():
        issue(ci + 2, 0)

      drain(ci + 1, 1)
      compute(ci + 1, 1)

    pltpu.sync_copy(pos_buf, pos_hbm.at[pl.ds(base, PER_W)])
    pltpu.sync_copy(neg_buf, neg_hbm.at[pl.ds(base, PER_W)])

  cp = pltpu.CompilerParams()
  fields = getattr(pltpu.CompilerParams, "__dataclass_fields__", {})
  if "needs_layout_passes" in fields:
    cp = dataclasses.replace(cp, needs_layout_passes=False)
  if "use_tc_tiling_on_sc" in fields:
    cp = dataclasses.replace(cp, use_tc_tiling_on_sc=False)

  return pl.kernel(
      body,
      out_type=(jax.ShapeDtypeStruct((B,), dtype),
                jax.ShapeDtypeStruct((B, NEG), dtype)),
      mesh=mesh,
      compiler_params=cp,
      scratch_types=[
          pltpu.VMEM((PER_W * CTX,), jnp.int32),
          pltpu.VMEM((PER_W,), jnp.int32),
          pltpu.VMEM((PER_W * NEG,), jnp.int32),
          pltpu.VMEM((2, C * CTX, D), dtype),
          pltpu.VMEM((2, C, D), dtype),
          pltpu.VMEM((2, C * NEG, D), dtype),
          pltpu.VMEM((D * LANES,), dtype),
          pltpu.VMEM((PER_W,), dtype),
          pltpu.VMEM((PER_W, NEG), dtype),
          pltpu.SemaphoreType.DMA((2,)),
      ],
  )


def kernel(input_ids, labels, negative_samples, emb_in, emb_out):
  B, CTX = input_ids.shape
  NEG = negative_samples.shape[1]
  D = emb_in.shape[1]
  ctx_idx = input_ids.reshape(-1).astype(jnp.int32)
  tgt_idx = labels.reshape(-1).astype(jnp.int32)
  neg_idx = negative_samples.reshape(-1).astype(jnp.int32)
  call = _make_sc_call(B, CTX, NEG, D, emb_in.dtype)
  return call(ctx_idx, tgt_idx, neg_idx, emb_in, emb_out)


# speedup vs baseline: 5.4267x; 1.5802x over previous
import dataclasses
import functools

import jax
import jax.numpy as jnp
from jax import lax
from jax.experimental import pallas as pl
from jax.experimental.pallas import tpu as pltpu
from jax.experimental.pallas import tpu_sc as plsc

LANES = 16
STREAM_IDX = 128


def _tree_sum(vals):
  vals = list(vals)
  while len(vals) > 1:
    nxt = [a + b for a, b in zip(vals[0::2], vals[1::2])]
    if len(vals) % 2:
      nxt.append(vals[-1])
    vals = nxt
  return vals[0]


def _make_sc_call(B, CTX, NEG, D, dtype):
  mesh = plsc.VectorSubcoreMesh(core_axis_name="c", subcore_axis_name="s")
  NC = mesh.num_cores
  NW = NC * mesh.num_subcores
  assert B % (NW * LANES) == 0
  PER_W = B // NW
  C = LANES
  NCHUNK = PER_W // C
  assert NCHUNK % 2 == 0

  def body(ctx_idx_hbm, tgt_idx_hbm, neg_idx_hbm, emb_in_hbm, emb_out_hbm,
           pos_hbm, neg_hbm,
           ctx_idx_v, tgt_idx_v, neg_idx_v,
           ctx_rows, tgt_rows, neg_rows,
           mean_v, pos_buf, neg_buf, sems):
    cid = lax.axis_index("c")
    sid = lax.axis_index("s")
    wid = sid * NC + cid
    base = wid * PER_W

    pltpu.sync_copy(ctx_idx_hbm.at[pl.ds(base * CTX, PER_W * CTX)], ctx_idx_v)
    pltpu.sync_copy(tgt_idx_hbm.at[pl.ds(base, PER_W)], tgt_idx_v)
    pltpu.sync_copy(neg_idx_hbm.at[pl.ds(base * NEG, PER_W * NEG)], neg_idx_v)

    e_iota = lax.iota(jnp.int32, LANES)

    def stream_pieces(n):
      pieces, o = [], 0
      while o < n:
        w = min(STREAM_IDX, n - o)
        pieces.append((o, w))
        o += w
      return pieces

    def gather_descs(ci, b):
      descs = []
      co = ci * C
      for o, w in stream_pieces(C * CTX):
        descs.append(pltpu.make_async_copy(
            emb_in_hbm.at[ctx_idx_v.at[pl.ds(co * CTX + o, w)]],
            ctx_rows.at[b].at[pl.ds(o, w)], sems.at[b]))
      for o, w in stream_pieces(C * NEG):
        descs.append(pltpu.make_async_copy(
            emb_out_hbm.at[neg_idx_v.at[pl.ds(co * NEG + o, w)]],
            neg_rows.at[b].at[pl.ds(o, w)], sems.at[b]))
      descs.append(pltpu.make_async_copy(
          emb_out_hbm.at[tgt_idx_v.at[pl.ds(co, C)]],
          tgt_rows.at[b], sems.at[b]))
      return descs

    def issue(ci, b):
      for d_ in gather_descs(ci, b):
        d_.start()

    def drain(ci, b):
      for d_ in gather_descs(ci, b):
        d_.wait()

    row_ctx = [e_iota * CTX + j for j in range(CTX)]
    row_neg = [e_iota * NEG + n for n in range(NEG)]

    def compute(ci, b):
      co = ci * C
      crows = ctx_rows.at[b]
      nrows = neg_rows.at[b]
      trows = tgt_rows.at[b]


      @pl.loop(0, D)
      def _mean(d):
        cold = (e_iota + d) & (D - 1)
        m = _tree_sum([plsc.load_gather(crows, [row_ctx[j], cold])
                       for j in range(CTX)])
        mean_v[pl.ds(d * LANES, LANES)] = m * (1.0 / CTX)

      def dbody(d, carry):
        pos_acc, neg_accs = carry
        cold = (e_iota + d) & (D - 1)
        m = mean_v[pl.ds(d * LANES, LANES)]
        pos_acc = pos_acc + plsc.load_gather(trows, [e_iota, cold]) * m
        neg_accs = tuple(
            neg_accs[n] + plsc.load_gather(nrows, [row_neg[n], cold]) * m
            for n in range(NEG))
        return pos_acc, neg_accs

      zero = jnp.zeros((LANES,), jnp.float32)
      pos_acc, neg_accs = lax.fori_loop(0, D, dbody, (zero, (zero,) * NEG))

      pos_buf[pl.ds(co, LANES)] = pos_acc
      for n in range(NEG):
        plsc.store_scatter(neg_buf,
                           [co + e_iota, jnp.full((LANES,), n, jnp.int32)],
                           neg_accs[n])

    issue(0, 0)

    @pl.loop(0, NCHUNK, step=2)
    def _chunk(ci):
      issue(ci + 1, 1)
      drain(ci, 0)
      compute(ci, 0)

      @pl.when(ci + 2 < NCHUNK)
      def _():
        issue(ci + 2, 0)

      drain(ci + 1, 1)
      compute(ci + 1, 1)

    pltpu.sync_copy(pos_buf, pos_hbm.at[pl.ds(base, PER_W)])
    pltpu.sync_copy(neg_buf, neg_hbm.at[pl.ds(base, PER_W)])

  cp = pltpu.CompilerParams()
  fields = getattr(pltpu.CompilerParams, "__dataclass_fields__", {})
  if "needs_layout_passes" in fields:
    cp = dataclasses.replace(cp, needs_layout_passes=False)
  if "use_tc_tiling_on_sc" in fields:
    cp = dataclasses.replace(cp, use_tc_tiling_on_sc=False)

  return pl.kernel(
      body,
      out_type=(jax.ShapeDtypeStruct((B,), dtype),
                jax.ShapeDtypeStruct((B, NEG), dtype)),
      mesh=mesh,
      compiler_params=cp,
      scratch_types=[
          pltpu.VMEM((PER_W * CTX,), jnp.int32),
          pltpu.VMEM((PER_W,), jnp.int32),
          pltpu.VMEM((PER_W * NEG,), jnp.int32),
          pltpu.VMEM((2, C * CTX, D), dtype),
          pltpu.VMEM((2, C, D), dtype),
          pltpu.VMEM((2, C * NEG, D), dtype),
          pltpu.VMEM((D * LANES,), dtype),
          pltpu.VMEM((PER_W,), dtype),
          pltpu.VMEM((PER_W, NEG), dtype),
          pltpu.SemaphoreType.DMA((2,)),
      ],
  )


def kernel(input_ids, labels, negative_samples, emb_in, emb_out):
  B, CTX = input_ids.shape
  NEG = negative_samples.shape[1]
  D = emb_in.shape[1]
  ctx_idx = input_ids.reshape(-1).astype(jnp.int32)
  tgt_idx = labels.reshape(-1).astype(jnp.int32)
  neg_idx = negative_samples.reshape(-1).astype(jnp.int32)
  call = _make_sc_call(B, CTX, NEG, D, emb_in.dtype)
  return call(ctx_idx, tgt_idx, neg_idx, emb_in, emb_out)
